# baseline (speedup 1.0000x reference)
import functools

import jax
import jax.numpy as jnp
from jax import lax
from jax.experimental import pallas as pl
from jax.experimental.pallas import tpu as pltpu
from jax.experimental.pallas import tpu_sc as plsc

N = 10000
E = 320000
D = 128
DE = 16
H = 128
DEPTH = 3

NC = 2
NS = 16
NW = NC * NS
CHUNK = 128
ACC = 10112
RPT = ACC // NS
TRASH = N + 8

NCH_H = 80
GC_H = 16
EP = CHUNK * NCH_H * NW

EH = E // 2
NCH_E = 40
GC_E = 8
EPH = CHUNK * NCH_E * NW


def _gs_body(nch, gc, table, src_i, dst_i, init, out, src_v, dst_v, rows_v,
             acc, sem0, sem1):
    c = lax.axis_index("c")
    s = lax.axis_index("s")
    wid = c * NS + s
    pltpu.sync_copy(init.at[c, pl.ds(s * RPT, RPT)], acc.at[pl.ds(s * RPT, RPT)])
    plsc.subcore_barrier()

    sems = (sem0, sem1)

    def stage(g):
        o = pl.multiple_of(g * gc, 8)
        pltpu.sync_copy(src_i.at[wid, pl.ds(o, gc)], src_v.at[g % 2])
        pltpu.sync_copy(dst_i.at[wid, pl.ds(o, gc)], dst_v.at[g % 2])

    def gather(k, b):
        g = k // gc
        pltpu.async_copy(table.at[src_v.at[g % 2, k % gc]], rows_v.at[b],
                         sems[b])

    stage(0)
    gather(0, 0)
    gather(1, 1)
    if nch > gc:
        stage(1)

    def body(j, carry):
        for b in range(2):
            k = 2 * j + b
            g = k // gc
            pltpu.make_async_copy(table.at[src_v.at[g % 2, k % gc]],
                                  rows_v.at[b], sems[b]).wait()
            pltpu.sync_copy(rows_v.at[b], acc.at[dst_v.at[g % 2, k % gc]],
                            add=True)

            kn = k + 2
            @pl.when(kn < nch)
            def _():
                gn = kn // gc
                pltpu.async_copy(table.at[src_v.at[gn % 2, kn % gc]],
                                 rows_v.at[b], sems[b])

            @pl.when((k % gc == 2) & (g >= 1) & (g + 1 < nch // gc))
            def _():
                stage(g + 1)
        return carry

    lax.fori_loop(0, nch // 2, body, 0)
    plsc.subcore_barrier()
    pltpu.sync_copy(acc.at[pl.ds(s * RPT, RPT)],
                    out.at[c, pl.ds(s * RPT, RPT)])


def _make_gs(nch, gc):
    mesh = plsc.VectorSubcoreMesh(core_axis_name="c", subcore_axis_name="s")
    return pl.kernel(
        functools.partial(_gs_body, nch, gc),
        out_type=jax.ShapeDtypeStruct((NC, ACC, H), jnp.float32),
        mesh=mesh,
        scratch_types=[
            pltpu.VMEM((2, gc, CHUNK), jnp.int32),
            pltpu.VMEM((2, gc, CHUNK), jnp.int32),
            pltpu.VMEM((2, CHUNK, H), jnp.float32),
            pltpu.VMEM_SHARED((ACC, H), jnp.float32),
            pltpu.SemaphoreType.DMA,
            pltpu.SemaphoreType.DMA,
        ],
    )


def _mm_relu_body(x_ref, w_ref, o_ref):
    o_ref[...] = jax.nn.relu(
        jnp.dot(x_ref[...], w_ref[...], preferred_element_type=jnp.float32))


def _mm_relu(x, w, blk, out_rows=None):
    m, k = x.shape
    _, n = w.shape
    return pl.pallas_call(
        _mm_relu_body,
        grid=(m // blk,),
        in_specs=[
            pl.BlockSpec((blk, k), lambda i: (i, 0)),
            pl.BlockSpec((k, n), lambda i: (0, 0)),
        ],
        out_specs=pl.BlockSpec((blk, n), lambda i: (i, 0)),
        out_shape=jax.ShapeDtypeStruct((out_rows or m, n), jnp.float32),
    )(x, w)


def _upd_body(h_ref, ph_ref, pe_ref, w_ref, h_out):
    m = ph_ref[0] + ph_ref[1] + pe_ref[0] + pe_ref[1]
    h_out[...] = jax.nn.relu(
        jnp.dot(h_ref[...], w_ref[...], preferred_element_type=jnp.float32) + m)


def _update(h, parts_h, parts_e, w, blk=2000):
    return pl.pallas_call(
        _upd_body,
        grid=(N // blk,),
        in_specs=[
            pl.BlockSpec((blk, H), lambda i: (i, 0)),
            pl.BlockSpec((NC, blk, H), lambda i: (0, i, 0)),
            pl.BlockSpec((NC, blk, H), lambda i: (0, i, 0)),
            pl.BlockSpec((H, H), lambda i: (0, 0)),
        ],
        out_specs=pl.BlockSpec((blk, H), lambda i: (i, 0)),
        out_shape=jax.ShapeDtypeStruct((N, H), jnp.float32),
    )(h, parts_h, parts_e, w)


def _updf_body(h_ref, ph_ref, pe_ref, w_ref, w1_ref, b1_ref, w2_ref, b2_ref,
               h_out, m_out, r_out):
    m = ph_ref[0] + ph_ref[1] + pe_ref[0] + pe_ref[1]
    m_out[...] = m
    hn = jax.nn.relu(
        jnp.dot(h_ref[...], w_ref[...], preferred_element_type=jnp.float32) + m)
    h_out[...] = hn
    a = jax.nn.relu(
        jnp.dot(hn, w1_ref[...], preferred_element_type=jnp.float32)
        + b1_ref[...])
    r_out[...] = (jnp.dot(a, w2_ref[...], preferred_element_type=jnp.float32)
                  + b2_ref[...])


def _update_final(h, parts_h, parts_e, w, w1, b1, w2, b2, blk=2000):
    ffn_h = w1.shape[1]
    out = w2.shape[1]
    return pl.pallas_call(
        _updf_body,
        grid=(N // blk,),
        in_specs=[
            pl.BlockSpec((blk, H), lambda i: (i, 0)),
            pl.BlockSpec((NC, blk, H), lambda i: (0, i, 0)),
            pl.BlockSpec((NC, blk, H), lambda i: (0, i, 0)),
            pl.BlockSpec((H, H), lambda i: (0, 0)),
            pl.BlockSpec((H, ffn_h), lambda i: (0, 0)),
            pl.BlockSpec((1, ffn_h), lambda i: (0, 0)),
            pl.BlockSpec((ffn_h, out), lambda i: (0, 0)),
            pl.BlockSpec((1, out), lambda i: (0, 0)),
        ],
        out_specs=[
            pl.BlockSpec((blk, H), lambda i: (i, 0)),
            pl.BlockSpec((blk, H), lambda i: (i, 0)),
            pl.BlockSpec((blk, out), lambda i: (i, 0)),
        ],
        out_shape=[
            jax.ShapeDtypeStruct((N, H), jnp.float32),
            jax.ShapeDtypeStruct((N, H), jnp.float32),
            jax.ShapeDtypeStruct((N, out), jnp.float32),
        ],
    )(h, parts_h, parts_e, w, w1, b1.reshape(1, ffn_h), w2,
      b2.reshape(1, out))


def kernel(x, edge_attr, W_in, W_h, W_e, ffn_W1, ffn_b1, ffn_W2, ffn_b2,
           edge_index):
    src = edge_index[0]
    dst = edge_index[1]
    pad_h = EP - E
    pad_e = EPH - EH
    src_p = jnp.concatenate(
        [src, jnp.arange(pad_h, dtype=jnp.int32) % N]).reshape(NW, NCH_H, CHUNK)
    dst_p = jnp.concatenate(
        [dst, jnp.full((pad_h,), TRASH, jnp.int32)]).reshape(NW, NCH_H, CHUNK)
    iota_e = jnp.concatenate(
        [jnp.arange(EH, dtype=jnp.int32),
         jnp.arange(pad_e, dtype=jnp.int32)]).reshape(NW, NCH_E, CHUNK)
    trash_e = jnp.full((pad_e,), TRASH, jnp.int32)
    dst_e0 = jnp.concatenate([dst[:EH], trash_e]).reshape(NW, NCH_E, CHUNK)
    dst_e1 = jnp.concatenate([dst[EH:], trash_e]).reshape(NW, NCH_E, CHUNK)

    h = _mm_relu(x, W_in, blk=2000)
    ea16 = edge_attr.astype(jnp.bfloat16)
    we16 = W_e.astype(jnp.bfloat16)
    e_p0 = _mm_relu(ea16[:EH], we16, blk=3200, out_rows=EPH)
    e_p1 = _mm_relu(ea16[EH:], we16, blk=3200, out_rows=EPH)

    zero_init = jnp.zeros((NC, ACC, H), jnp.float32)
    gs_e = _make_gs(NCH_E, GC_E)
    gs_h = _make_gs(NCH_H, GC_H)
    parts_h = gs_h(h, src_p, dst_p, zero_init)
    e_p0, _ = lax.optimization_barrier((e_p0, parts_h))
    parts_e = gs_e(e_p0, iota_e, dst_e0, zero_init)
    parts_e = gs_e(e_p1, iota_e, dst_e1, parts_e)

    h = _update(h, parts_h, parts_e, W_h)
    for _ in range(DEPTH - 2):
        parts_h = gs_h(h, src_p, dst_p, zero_init)
        h = _update(h, parts_h, parts_e, W_h)
    parts_h = gs_h(h, src_p, dst_p, zero_init)
    h, m, r = _update_final(h, parts_h, parts_e, W_h, ffn_W1, ffn_b1,
                            ffn_W2, ffn_b2)
    return (r, m, h)


# speedup vs baseline: 1.0733x; 1.0733x over previous
import functools

import jax
import jax.numpy as jnp
from jax import lax
from jax.experimental import pallas as pl
from jax.experimental.pallas import tpu as pltpu
from jax.experimental.pallas import tpu_sc as plsc

N = 10000
E = 320000
D = 128
DE = 16
H = 128
DEPTH = 3

NC = 2
NS = 16
NW = NC * NS
CHUNK = 128
ACC = 10112
RPT = ACC // NS
TRASH = N + 8

NCH_H = 80
GC_H = 16
EP = CHUNK * NCH_H * NW

EH = E // 2
NCH_E = 40
GC_E = 8
EPH = CHUNK * NCH_E * NW


def _gs_body(nch, gc, table, src_i, dst_i, init, out, src_v, dst_v, rows_v,
             acc, sem0, sem1):
    c = lax.axis_index("c")
    s = lax.axis_index("s")
    wid = c * NS + s
    pltpu.sync_copy(init.at[c, pl.ds(s * RPT, RPT)], acc.at[pl.ds(s * RPT, RPT)])
    plsc.subcore_barrier()

    sems = (sem0, sem1)

    def stage(g):
        o = pl.multiple_of(g * gc, 8)
        pltpu.sync_copy(src_i.at[wid, pl.ds(o, gc)], src_v.at[g % 2])
        pltpu.sync_copy(dst_i.at[wid, pl.ds(o, gc)], dst_v.at[g % 2])

    def gather(k, b):
        g = k // gc
        pltpu.async_copy(table.at[src_v.at[g % 2, k % gc]], rows_v.at[b],
                         sems[b])

    stage(0)
    gather(0, 0)
    gather(1, 1)
    if nch > gc:
        stage(1)

    def body(j, carry):
        for b in range(2):
            k = 2 * j + b
            g = k // gc
            pltpu.make_async_copy(table.at[src_v.at[g % 2, k % gc]],
                                  rows_v.at[b], sems[b]).wait()
            pltpu.sync_copy(rows_v.at[b], acc.at[dst_v.at[g % 2, k % gc]],
                            add=True)

            kn = k + 2
            @pl.when(kn < nch)
            def _():
                gn = kn // gc
                pltpu.async_copy(table.at[src_v.at[gn % 2, kn % gc]],
                                 rows_v.at[b], sems[b])

            @pl.when((k % gc == 2) & (g >= 1) & (g + 1 < nch // gc))
            def _():
                stage(g + 1)
        return carry

    lax.fori_loop(0, nch // 2, body, 0)
    plsc.subcore_barrier()
    pltpu.sync_copy(acc.at[pl.ds(s * RPT, RPT)],
                    out.at[c, pl.ds(s * RPT, RPT)])


def _make_gs(nch, gc):
    mesh = plsc.VectorSubcoreMesh(core_axis_name="c", subcore_axis_name="s")
    return pl.kernel(
        functools.partial(_gs_body, nch, gc),
        out_type=jax.ShapeDtypeStruct((NC, ACC, H), jnp.float32),
        mesh=mesh,
        scratch_types=[
            pltpu.VMEM((2, gc, CHUNK), jnp.int32),
            pltpu.VMEM((2, gc, CHUNK), jnp.int32),
            pltpu.VMEM((2, CHUNK, H), jnp.float32),
            pltpu.VMEM_SHARED((ACC, H), jnp.float32),
            pltpu.SemaphoreType.DMA,
            pltpu.SemaphoreType.DMA,
        ],
    )


def _mm_relu_body(x_ref, w_ref, o_ref):
    o_ref[...] = jax.nn.relu(
        jnp.dot(x_ref[...], w_ref[...], preferred_element_type=jnp.float32))


def _mm_relu(x, w, blk, out_rows=None):
    m, k = x.shape
    _, n = w.shape
    return pl.pallas_call(
        _mm_relu_body,
        grid=(m // blk,),
        in_specs=[
            pl.BlockSpec((blk, k), lambda i: (i, 0)),
            pl.BlockSpec((k, n), lambda i: (0, 0)),
        ],
        out_specs=pl.BlockSpec((blk, n), lambda i: (i, 0)),
        out_shape=jax.ShapeDtypeStruct((out_rows or m, n), jnp.float32),
    )(x, w)


def _upd_body(h_ref, ph_ref, pe_ref, w_ref, h_out):
    m = ph_ref[0] + ph_ref[1] + pe_ref[0] + pe_ref[1]
    h_out[...] = jax.nn.relu(
        jnp.dot(h_ref[...], w_ref[...], preferred_element_type=jnp.float32) + m)


def _update(h, parts_h, parts_e, w, blk=2000):
    return pl.pallas_call(
        _upd_body,
        grid=(N // blk,),
        in_specs=[
            pl.BlockSpec((blk, H), lambda i: (i, 0)),
            pl.BlockSpec((NC, blk, H), lambda i: (0, i, 0)),
            pl.BlockSpec((NC, blk, H), lambda i: (0, i, 0)),
            pl.BlockSpec((H, H), lambda i: (0, 0)),
        ],
        out_specs=pl.BlockSpec((blk, H), lambda i: (i, 0)),
        out_shape=jax.ShapeDtypeStruct((N, H), jnp.float32),
    )(h, parts_h, parts_e, w)


def _updf_body(h_ref, ph_ref, pe_ref, w_ref, w1_ref, b1_ref, w2_ref, b2_ref,
               h_out, m_out, r_out):
    m = ph_ref[0] + ph_ref[1] + pe_ref[0] + pe_ref[1]
    m_out[...] = m
    hn = jax.nn.relu(
        jnp.dot(h_ref[...], w_ref[...], preferred_element_type=jnp.float32) + m)
    h_out[...] = hn
    a = jax.nn.relu(
        jnp.dot(hn, w1_ref[...], preferred_element_type=jnp.float32)
        + b1_ref[...])
    r_out[...] = (jnp.dot(a, w2_ref[...], preferred_element_type=jnp.float32)
                  + b2_ref[...])


def _update_final(h, parts_h, parts_e, w, w1, b1, w2, b2, blk=2000):
    ffn_h = w1.shape[1]
    out = w2.shape[1]
    return pl.pallas_call(
        _updf_body,
        grid=(N // blk,),
        in_specs=[
            pl.BlockSpec((blk, H), lambda i: (i, 0)),
            pl.BlockSpec((NC, blk, H), lambda i: (0, i, 0)),
            pl.BlockSpec((NC, blk, H), lambda i: (0, i, 0)),
            pl.BlockSpec((H, H), lambda i: (0, 0)),
            pl.BlockSpec((H, ffn_h), lambda i: (0, 0)),
            pl.BlockSpec((1, ffn_h), lambda i: (0, 0)),
            pl.BlockSpec((ffn_h, out), lambda i: (0, 0)),
            pl.BlockSpec((1, out), lambda i: (0, 0)),
        ],
        out_specs=[
            pl.BlockSpec((blk, H), lambda i: (i, 0)),
            pl.BlockSpec((blk, H), lambda i: (i, 0)),
            pl.BlockSpec((blk, out), lambda i: (i, 0)),
        ],
        out_shape=[
            jax.ShapeDtypeStruct((N, H), jnp.float32),
            jax.ShapeDtypeStruct((N, H), jnp.float32),
            jax.ShapeDtypeStruct((N, out), jnp.float32),
        ],
    )(h, parts_h, parts_e, w, w1, b1.reshape(1, ffn_h), w2,
      b2.reshape(1, out))


def kernel(x, edge_attr, W_in, W_h, W_e, ffn_W1, ffn_b1, ffn_W2, ffn_b2,
           edge_index):
    src = edge_index[0]
    dst = edge_index[1]
    pad_h = EP - E
    pad_e = EPH - EH
    src_p = jnp.concatenate(
        [src, jnp.arange(pad_h, dtype=jnp.int32) % N]).reshape(NW, NCH_H, CHUNK)
    dst_p = jnp.concatenate(
        [dst, jnp.full((pad_h,), TRASH, jnp.int32)]).reshape(NW, NCH_H, CHUNK)
    iota_p = jnp.concatenate(
        [jnp.arange(E, dtype=jnp.int32),
         jnp.arange(pad_h, dtype=jnp.int32)]).reshape(NW, NCH_H, CHUNK)

    h = _mm_relu(x, W_in, blk=2000)
    e_p = _mm_relu(edge_attr.astype(jnp.bfloat16),
                   W_e.astype(jnp.bfloat16), blk=3200, out_rows=EP)

    zero_init = jnp.zeros((NC, ACC, H), jnp.float32)
    gs_e = _make_gs(NCH_H, GC_H)
    gs_h = _make_gs(NCH_H, GC_H)
    parts_e = gs_e(e_p, iota_p, dst_p, zero_init)

    for _ in range(DEPTH - 1):
        parts_h = gs_h(h, src_p, dst_p, zero_init)
        h = _update(h, parts_h, parts_e, W_h)
    parts_h = gs_h(h, src_p, dst_p, zero_init)
    h, m, r = _update_final(h, parts_h, parts_e, W_h, ffn_W1, ffn_b1,
                            ffn_W2, ffn_b2)
    return (r, m, h)
